# Initial kernel scaffold; baseline (speedup 1.0000x reference)
import jax
import jax.numpy as jnp
from jax.experimental import pallas as pl


def kernel(xyz1, f1, xyz2, f2, motion, W_down1, b_down1, W_res, b_res, Wm1, bm1, Wm2, bm2):
    raise NotImplementedError("write your pallas kernel here")


# speedup vs baseline: 44.3345x; 44.3345x over previous
import functools

import jax
import jax.numpy as jnp
from jax import lax
from jax.experimental import pallas as pl
from jax.experimental.pallas import tpu as pltpu
from jax.experimental.pallas import tpu_sc as plsc

N1 = 1024
N2 = 1024
C = 64
NQ = N2 * C

def _top3_min(d2, nrows, ncols):
    iota = lax.broadcasted_iota(jnp.int32, (nrows, ncols), 1)
    d = d2
    mns, ids = [], []
    for _ in range(3):
        mn = jnp.min(d, axis=1, keepdims=True)
        idk = jnp.min(jnp.where(d == mn, iota, 2 ** 30), axis=1, keepdims=True)
        mns.append(mn)
        ids.append(idk)
        d = jnp.where(iota == idk, jnp.float32(jnp.inf), d)
    return mns, ids


def _kernel_a(x1t_ref, xyz2_ref, f1_ref, motion_ref, wd_ref, bd_ref,
              wm1_ref, bm1_ref, wm2_ref, bm2_ref, x2tile_ref,
              newf1_ref, point1_ref, qall_ref):
    x1t = x1t_ref[...]
    xyz2 = xyz2_ref[...]
    f1 = f1_ref[...]

    q2 = jnp.sum(xyz2 * xyz2, axis=1, keepdims=True)
    r2 = jnp.sum(x1t * x1t, axis=0, keepdims=True)
    d2 = q2 - 2.0 * jnp.dot(xyz2, x1t, preferred_element_type=jnp.float32) + r2
    _, ids = _top3_min(d2, N2, N1)
    iota = lax.broadcasted_iota(jnp.int32, (N2, N1), 1)
    inf = jnp.float32(jnp.inf)
    invs = []
    for k in range(3):
        mask_k = iota == ids[k]
        s2 = jnp.float32(0.0)
        for a in range(3):
            ra = lax.slice(x1t, (a, 0), (a + 1, N1))
            sa = jnp.min(jnp.where(mask_k, ra, inf), axis=1, keepdims=True)
            da = sa - lax.slice(xyz2, (0, a), (N2, a + 1))
            s2 = s2 + da * da
        dist = jnp.maximum(jnp.sqrt(s2), 1e-10)
        invs.append(1.0 / dist)
    norm = invs[0] + invs[1] + invs[2]
    wmat = jnp.zeros((N2, N1), jnp.float32)
    for k in range(3):
        wmat = wmat + jnp.where(iota == ids[k], invs[k] / norm, 0.0)
    newf1_ref[...] = lax.dot_general(
        wmat, f1, (((1,), (0,)), ((), ())),
        precision=lax.Precision.HIGHEST,
        preferred_element_type=jnp.float32)

    point1_ref[...] = (
        jnp.dot(f1, wd_ref[...], preferred_element_type=jnp.float32)
        + bd_ref[...])

    h = jnp.dot(motion_ref[...], wm1_ref[...],
                preferred_element_type=jnp.float32) + bm1_ref[...]
    h = jnp.maximum(h, 0.0)
    m = jnp.dot(h, wm2_ref[...], preferred_element_type=jnp.float32) + bm2_ref[...]
    qall_ref[...] = m + x2tile_ref[...]


def _kernel_b(q_ref, x1t_ref, lidx_ref, w_ref, *, bq):
    q = q_ref[...]
    x1t = x1t_ref[...]
    q2 = jnp.sum(q * q, axis=1, keepdims=True)
    r2 = jnp.sum(x1t * x1t, axis=0, keepdims=True)
    d2 = q2 - 2.0 * jnp.dot(q, x1t, preferred_element_type=jnp.float32) + r2
    mns, ids = _top3_min(d2, bq, N1)
    invs = []
    for k in range(3):
        dist2 = jnp.maximum(mns[k], 0.0) + 1e-8
        invs.append(1.0 / dist2)
    denom = jnp.maximum(invs[0] + invs[1] + invs[2], 3.0)
    ch = jnp.bitwise_and(
        lax.broadcasted_iota(jnp.int32, (bq, 1), 0), jnp.int32(C - 1))
    lidx_ref[...] = jnp.concatenate(
        [ids[k] * C + ch for k in range(3)], axis=1)
    w_ref[...] = jnp.concatenate([invs[k] / denom for k in range(3)], axis=1)


def _make_sc_gather():
    info = plsc.get_sparse_core_info()
    nw = info.num_cores * info.num_subcores
    qpw = NQ // nw
    mesh = plsc.VectorSubcoreMesh(core_axis_name="c", subcore_axis_name="s")

    @functools.partial(
        pl.kernel, mesh=mesh,
        out_type=jax.ShapeDtypeStruct((NQ,), jnp.float32),
        compiler_params=pltpu.CompilerParams(needs_layout_passes=False),
        scratch_types=[
            pltpu.VMEM((N1 * C,), jnp.float32),
            pltpu.VMEM((3, qpw), jnp.int32),
            pltpu.VMEM((3, qpw), jnp.float32),
            pltpu.VMEM((qpw,), jnp.float32),
        ])
    def sc_gather(p1_hbm, lidx_hbm, w_hbm, out_hbm, p1_v, li_v, w_v, out_v):
        wid = lax.axis_index("s") * info.num_cores + lax.axis_index("c")
        base = wid * qpw
        pltpu.sync_copy(p1_hbm, p1_v)
        pltpu.sync_copy(lidx_hbm.at[:, pl.ds(base, qpw)], li_v)
        pltpu.sync_copy(w_hbm.at[:, pl.ds(base, qpw)], w_v)

        def body(i, carry):
            off = i * 16
            acc = jnp.zeros((16,), jnp.float32)
            for k in range(3):
                lid = li_v[k, pl.ds(off, 16)]
                pt = plsc.load_gather(p1_v, [lid])
                wk = w_v[k, pl.ds(off, 16)]
                acc = acc + wk * pt
            out_v[pl.ds(off, 16)] = acc
            return carry

        lax.fori_loop(0, qpw // 16, body, 0)
        pltpu.sync_copy(out_v, out_hbm.at[pl.ds(base, qpw)])

    return sc_gather


def kernel(xyz1, f1, xyz2, f2, motion, W_down1, b_down1, W_res, b_res,
           Wm1, bm1, Wm2, bm2):
    del f2, W_res, b_res
    x1t = xyz1.T
    x2tile = jnp.tile(xyz2, (1, C))

    newf1, point1, qall = pl.pallas_call(
        _kernel_a,
        out_shape=[
            jax.ShapeDtypeStruct((N2, C), jnp.float32),
            jax.ShapeDtypeStruct((N1, C), jnp.float32),
            jax.ShapeDtypeStruct((N2, 3 * C), jnp.float32),
        ],
    )(x1t, xyz2, f1, motion, W_down1, b_down1.reshape(1, C),
      Wm1, bm1.reshape(1, 3 * C), Wm2, bm2.reshape(1, 3 * C), x2tile)

    bq = 2048
    grid = NQ // bq
    qflat = qall.reshape(NQ, 3)
    lidx, w = pl.pallas_call(
        functools.partial(_kernel_b, bq=bq),
        grid=(grid,),
        in_specs=[
            pl.BlockSpec((bq, 3), lambda i: (i, 0)),
            pl.BlockSpec((3, N1), lambda i: (0, 0)),
        ],
        out_specs=[
            pl.BlockSpec((bq, 3), lambda i: (i, 0)),
            pl.BlockSpec((bq, 3), lambda i: (i, 0)),
        ],
        out_shape=[
            jax.ShapeDtypeStruct((NQ, 3), jnp.int32),
            jax.ShapeDtypeStruct((NQ, 3), jnp.float32),
        ],
    )(qflat, x1t)

    sc_gather = _make_sc_gather()
    warped = sc_gather(point1.reshape(N1 * C), lidx.T, w.T)
    return warped.reshape(N2, C), newf1


# speedup vs baseline: 50.5635x; 1.1405x over previous
import functools

import jax
import jax.numpy as jnp
from jax import lax
from jax.experimental import pallas as pl
from jax.experimental.pallas import tpu as pltpu
from jax.experimental.pallas import tpu_sc as plsc

N1 = 1024
N2 = 1024
C = 64
NQ = N2 * C

def _top3_min(d2, nrows, ncols):
    iota = lax.broadcasted_iota(
        jnp.int32, (nrows, ncols), 1).astype(jnp.float32)
    d = d2
    mns, ids = [], []
    for k in range(3):
        mn = jnp.min(d, axis=1, keepdims=True)
        idk = jnp.min(jnp.where(d == mn, iota, jnp.float32(1e9)),
                      axis=1, keepdims=True)
        mns.append(mn)
        ids.append(idk.astype(jnp.int32))
        if k < 2:
            d = jnp.where(iota == idk, jnp.float32(jnp.inf), d)
    return mns, ids


def _kernel_a(x1t_ref, xyz2_ref, f1_ref, motion_ref, wd_ref, bd_ref,
              wm1_ref, bm1_ref, wm2_ref, bm2_ref, x2tile_ref,
              newf1_ref, point1_ref, qall_ref):
    x1t = x1t_ref[...]
    xyz2 = xyz2_ref[...]
    f1 = f1_ref[...]

    q2 = jnp.sum(xyz2 * xyz2, axis=1, keepdims=True)
    r2 = jnp.sum(x1t * x1t, axis=0, keepdims=True)
    d2 = q2 - 2.0 * jnp.dot(xyz2, x1t, preferred_element_type=jnp.float32) + r2
    _, ids = _top3_min(d2, N2, N1)
    iota = lax.broadcasted_iota(jnp.int32, (N2, N1), 1)
    inf = jnp.float32(jnp.inf)
    invs = []
    for k in range(3):
        mask_k = iota == ids[k]
        s2 = jnp.float32(0.0)
        for a in range(3):
            ra = lax.slice(x1t, (a, 0), (a + 1, N1))
            sa = jnp.min(jnp.where(mask_k, ra, inf), axis=1, keepdims=True)
            da = sa - lax.slice(xyz2, (0, a), (N2, a + 1))
            s2 = s2 + da * da
        dist = jnp.maximum(jnp.sqrt(s2), 1e-10)
        invs.append(1.0 / dist)
    norm = invs[0] + invs[1] + invs[2]
    wmat = jnp.zeros((N2, N1), jnp.float32)
    for k in range(3):
        wmat = wmat + jnp.where(iota == ids[k], invs[k] / norm, 0.0)
    newf1_ref[...] = lax.dot_general(
        wmat, f1, (((1,), (0,)), ((), ())),
        precision=lax.Precision.HIGHEST,
        preferred_element_type=jnp.float32)

    point1_ref[...] = (
        jnp.dot(f1, wd_ref[...], preferred_element_type=jnp.float32)
        + bd_ref[...])

    h = jnp.dot(motion_ref[...], wm1_ref[...],
                preferred_element_type=jnp.float32) + bm1_ref[...]
    h = jnp.maximum(h, 0.0)
    m = jnp.dot(h, wm2_ref[...], preferred_element_type=jnp.float32) + bm2_ref[...]
    qall_ref[...] = m + x2tile_ref[...]


def _kernel_b(q_ref, x1t_ref, lidx_ref, w_ref, *, bq):
    q = q_ref[...]
    x1t = x1t_ref[...]
    q2 = jnp.sum(q * q, axis=1, keepdims=True)
    r2 = jnp.sum(x1t * x1t, axis=0, keepdims=True)
    d2 = q2 - 2.0 * jnp.dot(q, x1t, preferred_element_type=jnp.float32) + r2
    mns, ids = _top3_min(d2, bq, N1)
    invs = []
    for k in range(3):
        dist2 = jnp.maximum(mns[k], 0.0) + 1e-8
        invs.append(1.0 / dist2)
    denom = jnp.maximum(invs[0] + invs[1] + invs[2], 3.0)
    ch = jnp.bitwise_and(
        lax.broadcasted_iota(jnp.int32, (bq, 1), 0), jnp.int32(C - 1))
    lidx_ref[...] = jnp.concatenate(
        [ids[k] * C + ch for k in range(3)], axis=1)
    w_ref[...] = jnp.concatenate([invs[k] / denom for k in range(3)], axis=1)


def _make_sc_gather():
    info = plsc.get_sparse_core_info()
    nw = info.num_cores * info.num_subcores
    qpw = NQ // nw
    mesh = plsc.VectorSubcoreMesh(core_axis_name="c", subcore_axis_name="s")

    @functools.partial(
        pl.kernel, mesh=mesh,
        out_type=jax.ShapeDtypeStruct((NQ,), jnp.float32),
        compiler_params=pltpu.CompilerParams(needs_layout_passes=False),
        scratch_types=[
            pltpu.VMEM((N1 * C,), jnp.float32),
            pltpu.VMEM((3, qpw), jnp.int32),
            pltpu.VMEM((3, qpw), jnp.float32),
            pltpu.VMEM((qpw,), jnp.float32),
        ])
    def sc_gather(p1_hbm, lidx_hbm, w_hbm, out_hbm, p1_v, li_v, w_v, out_v):
        wid = lax.axis_index("s") * info.num_cores + lax.axis_index("c")
        base = wid * qpw
        pltpu.sync_copy(p1_hbm, p1_v)
        pltpu.sync_copy(lidx_hbm.at[:, pl.ds(base, qpw)], li_v)
        pltpu.sync_copy(w_hbm.at[:, pl.ds(base, qpw)], w_v)

        def body(i, carry):
            off = i * 16
            acc = jnp.zeros((16,), jnp.float32)
            for k in range(3):
                lid = li_v[k, pl.ds(off, 16)]
                pt = plsc.load_gather(p1_v, [lid])
                wk = w_v[k, pl.ds(off, 16)]
                acc = acc + wk * pt
            out_v[pl.ds(off, 16)] = acc
            return carry

        lax.fori_loop(0, qpw // 16, body, 0)
        pltpu.sync_copy(out_v, out_hbm.at[pl.ds(base, qpw)])

    return sc_gather


def kernel(xyz1, f1, xyz2, f2, motion, W_down1, b_down1, W_res, b_res,
           Wm1, bm1, Wm2, bm2):
    del f2, W_res, b_res
    x1t = xyz1.T
    x2tile = jnp.tile(xyz2, (1, C))

    newf1, point1, qall = pl.pallas_call(
        _kernel_a,
        out_shape=[
            jax.ShapeDtypeStruct((N2, C), jnp.float32),
            jax.ShapeDtypeStruct((N1, C), jnp.float32),
            jax.ShapeDtypeStruct((N2, 3 * C), jnp.float32),
        ],
    )(x1t, xyz2, f1, motion, W_down1, b_down1.reshape(1, C),
      Wm1, bm1.reshape(1, 3 * C), Wm2, bm2.reshape(1, 3 * C), x2tile)

    bq = 2048
    grid = NQ // bq
    qflat = qall.reshape(NQ, 3)
    lidx, w = pl.pallas_call(
        functools.partial(_kernel_b, bq=bq),
        grid=(grid,),
        in_specs=[
            pl.BlockSpec((bq, 3), lambda i: (i, 0)),
            pl.BlockSpec((3, N1), lambda i: (0, 0)),
        ],
        out_specs=[
            pl.BlockSpec((bq, 3), lambda i: (i, 0)),
            pl.BlockSpec((bq, 3), lambda i: (i, 0)),
        ],
        out_shape=[
            jax.ShapeDtypeStruct((NQ, 3), jnp.int32),
            jax.ShapeDtypeStruct((NQ, 3), jnp.float32),
        ],
    )(qflat, x1t)

    sc_gather = _make_sc_gather()
    warped = sc_gather(point1.reshape(N1 * C), lidx.T, w.T)
    return warped.reshape(N2, C), newf1


# speedup vs baseline: 51.6227x; 1.0209x over previous
import functools

import jax
import jax.numpy as jnp
from jax import lax
from jax.experimental import pallas as pl
from jax.experimental.pallas import tpu as pltpu
from jax.experimental.pallas import tpu_sc as plsc

N1 = 1024
N2 = 1024
C = 64
NQ = N2 * C

def _top3_min(d2, nrows, ncols):
    iota = lax.broadcasted_iota(
        jnp.int32, (nrows, ncols), 1).astype(jnp.float32)
    d = d2
    mns, ids = [], []
    for k in range(3):
        mn = jnp.min(d, axis=1, keepdims=True)
        idk = jnp.min(jnp.where(d == mn, iota, jnp.float32(1e9)),
                      axis=1, keepdims=True)
        mns.append(mn)
        ids.append(idk.astype(jnp.int32))
        if k < 2:
            d = jnp.where(iota == idk, jnp.float32(jnp.inf), d)
    return mns, ids


def _kernel_a(x1t_ref, xyz2_ref, f1_ref, motion_ref, wd_ref, bd_ref,
              wm1_ref, bm1_ref, wm2_ref, bm2_ref, x2tile_ref,
              newf1_ref, point1_ref, qall_ref, x1aug_ref):
    x1t = x1t_ref[...]
    xyz2 = xyz2_ref[...]
    f1 = f1_ref[...]

    q2 = jnp.sum(xyz2 * xyz2, axis=1, keepdims=True)
    r2 = jnp.sum(x1t * x1t, axis=0, keepdims=True)
    d2 = q2 - 2.0 * jnp.dot(xyz2, x1t, preferred_element_type=jnp.float32) + r2
    x1aug_ref[...] = jnp.concatenate([-2.0 * x1t, r2], axis=0)
    _, ids = _top3_min(d2, N2, N1)
    iota = lax.broadcasted_iota(jnp.int32, (N2, N1), 1)
    inf = jnp.float32(jnp.inf)
    invs = []
    for k in range(3):
        mask_k = iota == ids[k]
        s2 = jnp.float32(0.0)
        for a in range(3):
            ra = lax.slice(x1t, (a, 0), (a + 1, N1))
            sa = jnp.min(jnp.where(mask_k, ra, inf), axis=1, keepdims=True)
            da = sa - lax.slice(xyz2, (0, a), (N2, a + 1))
            s2 = s2 + da * da
        dist = jnp.maximum(jnp.sqrt(s2), 1e-10)
        invs.append(1.0 / dist)
    norm = invs[0] + invs[1] + invs[2]
    wmat = jnp.zeros((N2, N1), jnp.float32)
    for k in range(3):
        wmat = wmat + jnp.where(iota == ids[k], invs[k] / norm, 0.0)
    newf1_ref[...] = lax.dot_general(
        wmat, f1, (((1,), (0,)), ((), ())),
        precision=lax.Precision.HIGHEST,
        preferred_element_type=jnp.float32)

    point1_ref[...] = (
        jnp.dot(f1, wd_ref[...], preferred_element_type=jnp.float32)
        + bd_ref[...])

    h = jnp.dot(motion_ref[...], wm1_ref[...],
                preferred_element_type=jnp.float32) + bm1_ref[...]
    h = jnp.maximum(h, 0.0)
    m = jnp.dot(h, wm2_ref[...], preferred_element_type=jnp.float32) + bm2_ref[...]
    qall_ref[...] = m + x2tile_ref[...]


def _kernel_b(q_ref, x1aug_ref, lidx_ref, w_ref, *, bq):
    q = q_ref[...]
    x1aug = x1aug_ref[...]
    q2 = jnp.sum(q * q, axis=1, keepdims=True)
    qpad = jnp.concatenate([q, jnp.ones((bq, 1), jnp.float32)], axis=1)
    s = jnp.dot(qpad, x1aug, preferred_element_type=jnp.float32)
    mns, ids = _top3_min(s, bq, N1)
    invs = []
    for k in range(3):
        dist2 = jnp.maximum(mns[k] + q2, 0.0) + 1e-8
        invs.append(1.0 / dist2)
    denom = jnp.maximum(invs[0] + invs[1] + invs[2], 3.0)
    ch = jnp.bitwise_and(
        lax.broadcasted_iota(jnp.int32, (bq, 1), 0), jnp.int32(C - 1))
    lidx_ref[...] = jnp.concatenate(
        [ids[k] * C + ch for k in range(3)], axis=1)
    w_ref[...] = jnp.concatenate([invs[k] / denom for k in range(3)], axis=1)


def _make_sc_gather():
    info = plsc.get_sparse_core_info()
    nw = info.num_cores * info.num_subcores
    qpw = NQ // nw
    mesh = plsc.VectorSubcoreMesh(core_axis_name="c", subcore_axis_name="s")

    @functools.partial(
        pl.kernel, mesh=mesh,
        out_type=jax.ShapeDtypeStruct((NQ,), jnp.float32),
        compiler_params=pltpu.CompilerParams(needs_layout_passes=False),
        scratch_types=[
            pltpu.VMEM((N1 * C,), jnp.float32),
            pltpu.VMEM((3, qpw), jnp.int32),
            pltpu.VMEM((3, qpw), jnp.float32),
            pltpu.VMEM((qpw,), jnp.float32),
        ])
    def sc_gather(p1_hbm, lidx_hbm, w_hbm, out_hbm, p1_v, li_v, w_v, out_v):
        wid = lax.axis_index("s") * info.num_cores + lax.axis_index("c")
        base = wid * qpw
        pltpu.sync_copy(p1_hbm, p1_v)
        pltpu.sync_copy(lidx_hbm.at[:, pl.ds(base, qpw)], li_v)
        pltpu.sync_copy(w_hbm.at[:, pl.ds(base, qpw)], w_v)

        def body(i, carry):
            off = i * 16
            acc = jnp.zeros((16,), jnp.float32)
            for k in range(3):
                lid = li_v[k, pl.ds(off, 16)]
                pt = plsc.load_gather(p1_v, [lid])
                wk = w_v[k, pl.ds(off, 16)]
                acc = acc + wk * pt
            out_v[pl.ds(off, 16)] = acc
            return carry

        lax.fori_loop(0, qpw // 16, body, 0)
        pltpu.sync_copy(out_v, out_hbm.at[pl.ds(base, qpw)])

    return sc_gather


def kernel(xyz1, f1, xyz2, f2, motion, W_down1, b_down1, W_res, b_res,
           Wm1, bm1, Wm2, bm2):
    del f2, W_res, b_res
    x1t = xyz1.T
    x2tile = jnp.tile(xyz2, (1, C))

    newf1, point1, qall, x1aug = pl.pallas_call(
        _kernel_a,
        out_shape=[
            jax.ShapeDtypeStruct((N2, C), jnp.float32),
            jax.ShapeDtypeStruct((N1, C), jnp.float32),
            jax.ShapeDtypeStruct((N2, 3 * C), jnp.float32),
            jax.ShapeDtypeStruct((4, N1), jnp.float32),
        ],
    )(x1t, xyz2, f1, motion, W_down1, b_down1.reshape(1, C),
      Wm1, bm1.reshape(1, 3 * C), Wm2, bm2.reshape(1, 3 * C), x2tile)

    bq = 2048
    grid = NQ // bq
    qflat = qall.reshape(NQ, 3)
    lidx, w = pl.pallas_call(
        functools.partial(_kernel_b, bq=bq),
        grid=(grid,),
        in_specs=[
            pl.BlockSpec((bq, 3), lambda i: (i, 0)),
            pl.BlockSpec((4, N1), lambda i: (0, 0)),
        ],
        out_specs=[
            pl.BlockSpec((bq, 3), lambda i: (i, 0)),
            pl.BlockSpec((bq, 3), lambda i: (i, 0)),
        ],
        out_shape=[
            jax.ShapeDtypeStruct((NQ, 3), jnp.int32),
            jax.ShapeDtypeStruct((NQ, 3), jnp.float32),
        ],
    )(qflat, x1aug)

    sc_gather = _make_sc_gather()
    warped = sc_gather(point1.reshape(N1 * C), lidx.T, w.T)
    return warped.reshape(N2, C), newf1
